# baseline (speedup 1.0000x reference)
import jax
import jax.numpy as jnp
from jax import lax
from jax.experimental import pallas as pl
from jax.experimental.pallas import tpu as pltpu
from jax.experimental.pallas import tpu_sc as plsc

N_CO = 100000
N_FACE = 10000
N_EDGE = 50000
E = 100000
BR = 512
E_PAD = 106496
E_ROWS = E_PAD // 128
ROWS_PER_TILE = E_ROWS // 16
NBATCH = ROWS_PER_TILE
SENTINEL = 1 << 30

N_CO_PAD = 100352
N_FACE_PAD = 10240
N_EDGE_PAD = 50176


def _make_agg(n_pad, n_passes, r_half):
  chunk = 2 * r_half
  r16 = r_half // 16
  n_wchunk = r16 // 16
  assert n_passes * chunk == n_pad and r_half % 256 == 0

  mesh = plsc.VectorSubcoreMesh(
      core_axis_name="c", subcore_axis_name="s", num_cores=2, num_subcores=16)

  def body(tab, src, dst, zfeat, zcnt, out, acc, cacc, rows_v, src_i, dst_i,
           dloc, obuf, cbuf, zbuf, zcb, ones_v, sem):
    cid = lax.axis_index("c")
    sid = lax.axis_index("s")
    pltpu.sync_copy(zfeat, zbuf)
    pltpu.sync_copy(zcnt, zcb)
    for f in range(8):
      ones_v[pl.ds(f * 16, 16)] = jnp.ones((16,), jnp.float32)

    def pass_body(p, carry):
      lo = p * chunk + cid * r_half
      for z in range(n_wchunk):
        pltpu.sync_copy(zbuf, acc.at[pl.ds(sid * r16 + z * 16, 16)])
      pltpu.sync_copy(zcb, cacc.at[pl.ds(sid * r16, r16)])
      plsc.subcore_barrier()

      def batch(b, c2):
        erow = sid * ROWS_PER_TILE + b
        pltpu.sync_copy(src.at[pl.ds(erow, 1)], src_i)
        pltpu.sync_copy(dst.at[pl.ds(erow, 1)], dst_i)
        for v in range(8):
          d = dst_i[0, pl.ds(v * 16, 16)]
          m = (d >= lo) & (d < lo + r_half)
          dummy = r_half + sid * 16 + 2 * v
          dloc[0, pl.ds(v * 16, 16)] = jnp.where(m, d - lo, dummy)
        pltpu.async_copy(tab.at[src_i.at[0]], rows_v, sem).wait()
        pltpu.sync_copy(rows_v, acc.at[dloc.at[0]], add=True)
        pltpu.sync_copy(ones_v, cacc.at[dloc.at[0]], add=True)
        return c2

      lax.fori_loop(0, NBATCH, batch, 0)
      plsc.subcore_barrier()

      def wchunk(ch, c3):
        rbase = sid * r16 + ch * 16
        pltpu.sync_copy(acc.at[pl.ds(rbase, 16)], obuf)
        pltpu.sync_copy(cacc.at[pl.ds(rbase, 16)], cbuf)
        rcp = 1.0 / jnp.clip(cbuf[pl.ds(0, 16)], 1.0, None)
        for r in range(16):
          scal = lax.gather(
              rcp, jnp.full((16, 1), r, jnp.int32),
              dimension_numbers=lax.GatherDimensionNumbers(
                  offset_dims=(), collapsed_slice_dims=(0,),
                  start_index_map=(0,)),
              slice_sizes=(1,),
              mode=lax.GatherScatterMode.PROMISE_IN_BOUNDS)
          for f in range(8):
            obuf[r, pl.ds(f * 16, 16)] = obuf[r, pl.ds(f * 16, 16)] * scal
        pltpu.sync_copy(obuf, out.at[pl.ds(lo + rbase, 16)])
        return c3

      lax.fori_loop(0, n_wchunk, wchunk, 0)
      plsc.subcore_barrier()
      return carry

    lax.fori_loop(0, n_passes, pass_body, 0)

  return pl.kernel(
      body,
      out_type=jax.ShapeDtypeStruct((n_pad, 128), jnp.float32),
      mesh=mesh,
      scratch_types=[
          pltpu.VMEM_SHARED((r_half + 256, 128), jnp.float32),
          pltpu.VMEM_SHARED((r_half + 256,), jnp.float32),
          pltpu.VMEM((128, 128), jnp.float32),
          pltpu.VMEM((1, 128), jnp.int32),
          pltpu.VMEM((1, 128), jnp.int32),
          pltpu.VMEM((1, 128), jnp.int32),
          pltpu.VMEM((16, 128), jnp.float32),
          pltpu.VMEM((16,), jnp.float32),
          pltpu.VMEM((16, 128), jnp.float32),
          pltpu.VMEM((r16,), jnp.float32),
          pltpu.VMEM((128,), jnp.float32),
          pltpu.SemaphoreType.DMA,
      ],
  )


def _dot(a, b):
  return jnp.dot(a, b, preferred_element_type=jnp.float32)


def _tc_two(mn, mm, x, wnl, wnr, wml, wmr, bn, bm, o):
  acc = (_dot(mn[:], wnl[:]) + _dot(mm[:], wml[:])
         + _dot(x[:], wnr[:] + wmr[:]) + bn[:] + bm[:])
  o[:] = jnp.maximum(acc, 0.0)


def _tc_one(s, x, wl, wr, b, o):
  acc = _dot(s[:], wl[:]) + _dot(x[:], wr[:]) + b[:]
  o[:] = jnp.maximum(acc, 0.0)


def _tc_two_ro(mn, mm, h, wnl, wnr, wml, wmr, bn, bm, wro, bro, o):
  g = jnp.maximum(
      _dot(mn[:], wnl[:]) + _dot(mm[:], wml[:])
      + _dot(h[:], wnr[:] + wmr[:]) + bn[:] + bm[:], 0.0)
  o[:] = _dot(g, wro[:]) + bro[:]


def _tc_one_ro(s, h, wl, wr, b, wro, bro, o):
  g = jnp.maximum(_dot(s[:], wl[:]) + _dot(h[:], wr[:]) + b[:], 0.0)
  o[:] = _dot(g, wro[:]) + bro[:]


def _row_spec(w):
  return pl.BlockSpec((BR, w), lambda i: (i, 0))


def _full_spec(shape):
  return pl.BlockSpec(shape, lambda i: (0, 0))


def _call_tc(body, n_pad, out_w, row_widths, full_shapes, args):
  grid = (n_pad // BR,)
  in_specs = ([_row_spec(w) for w in row_widths]
              + [_full_spec(s) for s in full_shapes])
  return pl.pallas_call(
      body,
      grid=grid,
      in_specs=in_specs,
      out_specs=_row_spec(out_w),
      out_shape=jax.ShapeDtypeStruct((n_pad, out_w), jnp.float32),
  )(*args)


def _pad_rows(x, n):
  return jnp.pad(x, ((0, n - x.shape[0]), (0, 0)))


def _pad_edges(ei):
  src = jnp.concatenate(
      [ei[0], jnp.zeros((E_PAD - E,), jnp.int32)]).reshape(E_ROWS, 128)
  dst = jnp.concatenate(
      [ei[1], jnp.full((E_PAD - E,), SENTINEL, jnp.int32)]).reshape(
          E_ROWS, 128)
  return src, dst


@jax.jit
def _run(x_coedge, x_face, x_edge, ei_next, ei_mate, ei_to_face, ei_to_edge,
         params):
  p = params
  xco = _pad_rows(x_coedge, N_CO_PAD)
  xfa = _pad_rows(x_face, N_FACE_PAD)
  xed = _pad_rows(x_edge, N_EDGE_PAD)
  e_next = _pad_edges(ei_next)
  e_mate = _pad_edges(ei_mate)
  e_face = _pad_edges(ei_to_face)
  e_edge = _pad_edges(ei_to_edge)
  zf = jnp.zeros((16, 128), jnp.float32)
  zc_co = jnp.zeros((784,), jnp.float32)
  zc_fa = jnp.zeros((320,), jnp.float32)

  agg_co = _make_agg(N_CO_PAD, 4, 12544)
  agg_fa = _make_agg(N_FACE_PAD, 1, 5120)
  agg_ed = _make_agg(N_EDGE_PAD, 2, 12544)

  def b2(name):
    return p[name].reshape(1, 128)

  m1n = agg_co(xco, *e_next, zf, zc_co)
  m1m = agg_co(xco, *e_mate, zf, zc_co)
  m1f = agg_fa(xco, *e_face, zf, zc_fa)
  m1e = agg_ed(xco, *e_edge, zf, zc_co)

  h_co = _call_tc(
      _tc_two, N_CO_PAD, 128, [128, 128, 128],
      [(128, 128)] * 4 + [(1, 128)] * 2,
      (m1n, m1m, xco, p['W1_next_l'], p['W1_next_r'], p['W1_mate_l'],
       p['W1_mate_r'], b2('b1_next'), b2('b1_mate')))
  h_fa = _call_tc(
      _tc_one, N_FACE_PAD, 128, [128, 128],
      [(128, 128)] * 2 + [(1, 128)],
      (m1f, xfa, p['W1_to_face_l'], p['W1_to_face_r'], b2('b1_to_face')))
  h_ed = _call_tc(
      _tc_one, N_EDGE_PAD, 128, [128, 128],
      [(128, 128)] * 2 + [(1, 128)],
      (m1e, xed, p['W1_to_edge_l'], p['W1_to_edge_r'], b2('b1_to_edge')))

  m2n = agg_co(h_co, *e_next, zf, zc_co)
  m2m = agg_co(h_co, *e_mate, zf, zc_co)
  m2f = agg_fa(h_co, *e_face, zf, zc_fa)
  m2e = agg_ed(h_co, *e_edge, zf, zc_co)

  z_co = _call_tc(
      _tc_two_ro, N_CO_PAD, 256, [128, 128, 128],
      [(128, 128)] * 4 + [(1, 128)] * 2 + [(128, 256), (1, 256)],
      (m2n, m2m, h_co, p['W2_next_l'], p['W2_next_r'], p['W2_mate_l'],
       p['W2_mate_r'], b2('b2_next'), b2('b2_mate'), p['Wro_coedge'],
       p['bro_coedge'].reshape(1, 256)))
  z_fa = _call_tc(
      _tc_one_ro, N_FACE_PAD, 256, [128, 128],
      [(128, 128)] * 2 + [(1, 128), (128, 256), (1, 256)],
      (m2f, h_fa, p['W2_to_face_l'], p['W2_to_face_r'], b2('b2_to_face'),
       p['Wro_face'], p['bro_face'].reshape(1, 256)))
  z_ed = _call_tc(
      _tc_one_ro, N_EDGE_PAD, 256, [128, 128],
      [(128, 128)] * 2 + [(1, 128), (128, 256), (1, 256)],
      (m2e, h_ed, p['W2_to_edge_l'], p['W2_to_edge_r'], b2('b2_to_edge'),
       p['Wro_edge'], p['bro_edge'].reshape(1, 256)))

  return (z_co[:N_CO], z_fa[:N_FACE], z_ed[:N_EDGE])


def kernel(x_coedge, x_face, x_edge, ei_next, ei_mate, ei_to_face,
           ei_to_edge, params):
  return _run(x_coedge, x_face, x_edge, ei_next, ei_mate, ei_to_face,
              ei_to_edge, params)


# speedup vs baseline: 1.0098x; 1.0098x over previous
import jax
import jax.numpy as jnp
from jax import lax
from jax.experimental import pallas as pl
from jax.experimental.pallas import tpu as pltpu
from jax.experimental.pallas import tpu_sc as plsc

N_CO = 100000
N_FACE = 10000
N_EDGE = 50000
E = 100000
BR = 512
E_PAD = 106496
E_ROWS = E_PAD // 128
ROWS_PER_TILE = E_ROWS // 16
NBATCH = ROWS_PER_TILE
SENTINEL = 1 << 30

N_CO_PAD = 100352
N_FACE_PAD = 10240
N_EDGE_PAD = 50176


def _make_agg(n_pad, n_passes, r_half):
  chunk = 2 * r_half
  r16 = r_half // 16
  n_wchunk = r16 // 16
  assert n_passes * chunk == n_pad and r_half % 256 == 0

  mesh = plsc.VectorSubcoreMesh(
      core_axis_name="c", subcore_axis_name="s", num_cores=2, num_subcores=16)

  def body(tab, src, dst, zfeat, zcnt, out, acc, cacc, rows_v, src_i, dst_i,
           dloc, obuf, cbuf, zbuf, zcb, ones_v, *sems):
    cid = lax.axis_index("c")
    sid = lax.axis_index("s")
    pltpu.sync_copy(zfeat, zbuf)
    pltpu.sync_copy(zcnt, zcb)
    for f in range(8):
      ones_v[pl.ds(f * 16, 16)] = jnp.ones((16,), jnp.float32)

    def pass_body(p, carry):
      lo = p * chunk + cid * r_half
      for z in range(n_wchunk):
        pltpu.sync_copy(zbuf, acc.at[pl.ds(sid * r16 + z * 16, 16)])
      pltpu.sync_copy(zcb, cacc.at[pl.ds(sid * r16, r16)])
      plsc.subcore_barrier()

      def batch(b, c2):
        erow = sid * ROWS_PER_TILE + b * 4
        pltpu.sync_copy(src.at[pl.ds(erow, 4)], src_i)
        pltpu.sync_copy(dst.at[pl.ds(erow, 4)], dst_i)
        for jb in range(4):
          for v in range(8):
            d = dst_i[jb, pl.ds(v * 16, 16)]
            m = (d >= lo) & (d < lo + r_half)
            dummy = r_half + sid * 16 + 2 * v
            dloc[jb, pl.ds(v * 16, 16)] = jnp.where(m, d - lo, dummy)
          descs = [
              pltpu.async_copy(tab.at[src_i.at[jb][pl.ds(u * 16, 16)]],
                               rows_v.at[pl.ds(u * 16, 16)], sems[u])
              for u in range(8)
          ]
          for dsc in descs:
            dsc.wait()
          pltpu.sync_copy(rows_v, acc.at[dloc.at[jb]], add=True)
          pltpu.sync_copy(ones_v, cacc.at[dloc.at[jb]], add=True)
        return c2

      lax.fori_loop(0, NBATCH // 4, batch, 0)
      plsc.subcore_barrier()

      def wchunk(ch, c3):
        rbase = sid * r16 + ch * 16
        pltpu.sync_copy(acc.at[pl.ds(rbase, 16)], obuf)
        pltpu.sync_copy(cacc.at[pl.ds(rbase, 16)], cbuf)
        rcp = 1.0 / jnp.clip(cbuf[pl.ds(0, 16)], 1.0, None)
        for r in range(16):
          scal = lax.gather(
              rcp, jnp.full((16, 1), r, jnp.int32),
              dimension_numbers=lax.GatherDimensionNumbers(
                  offset_dims=(), collapsed_slice_dims=(0,),
                  start_index_map=(0,)),
              slice_sizes=(1,),
              mode=lax.GatherScatterMode.PROMISE_IN_BOUNDS)
          for f in range(8):
            obuf[r, pl.ds(f * 16, 16)] = obuf[r, pl.ds(f * 16, 16)] * scal
        pltpu.sync_copy(obuf, out.at[pl.ds(lo + rbase, 16)])
        return c3

      lax.fori_loop(0, n_wchunk, wchunk, 0)
      plsc.subcore_barrier()
      return carry

    lax.fori_loop(0, n_passes, pass_body, 0)

  return pl.kernel(
      body,
      out_type=jax.ShapeDtypeStruct((n_pad, 128), jnp.float32),
      mesh=mesh,
      scratch_types=[
          pltpu.VMEM_SHARED((r_half + 256, 128), jnp.float32),
          pltpu.VMEM_SHARED((r_half + 256,), jnp.float32),
          pltpu.VMEM((128, 128), jnp.float32),
          pltpu.VMEM((4, 128), jnp.int32),
          pltpu.VMEM((4, 128), jnp.int32),
          pltpu.VMEM((4, 128), jnp.int32),
          pltpu.VMEM((16, 128), jnp.float32),
          pltpu.VMEM((16,), jnp.float32),
          pltpu.VMEM((16, 128), jnp.float32),
          pltpu.VMEM((r16,), jnp.float32),
          pltpu.VMEM((128,), jnp.float32),
      ] + [pltpu.SemaphoreType.DMA] * 8,
  )


def _dot(a, b):
  return jnp.dot(a, b, preferred_element_type=jnp.float32)


def _tc_two(mn, mm, x, wnl, wnr, wml, wmr, bn, bm, o):
  acc = (_dot(mn[:], wnl[:]) + _dot(mm[:], wml[:])
         + _dot(x[:], wnr[:] + wmr[:]) + bn[:] + bm[:])
  o[:] = jnp.maximum(acc, 0.0)


def _tc_one(s, x, wl, wr, b, o):
  acc = _dot(s[:], wl[:]) + _dot(x[:], wr[:]) + b[:]
  o[:] = jnp.maximum(acc, 0.0)


def _tc_two_ro(mn, mm, h, wnl, wnr, wml, wmr, bn, bm, wro, bro, o):
  g = jnp.maximum(
      _dot(mn[:], wnl[:]) + _dot(mm[:], wml[:])
      + _dot(h[:], wnr[:] + wmr[:]) + bn[:] + bm[:], 0.0)
  o[:] = _dot(g, wro[:]) + bro[:]


def _tc_one_ro(s, h, wl, wr, b, wro, bro, o):
  g = jnp.maximum(_dot(s[:], wl[:]) + _dot(h[:], wr[:]) + b[:], 0.0)
  o[:] = _dot(g, wro[:]) + bro[:]


def _row_spec(w):
  return pl.BlockSpec((BR, w), lambda i: (i, 0))


def _full_spec(shape):
  return pl.BlockSpec(shape, lambda i: (0, 0))


def _call_tc(body, n_pad, out_w, row_widths, full_shapes, args):
  grid = (n_pad // BR,)
  in_specs = ([_row_spec(w) for w in row_widths]
              + [_full_spec(s) for s in full_shapes])
  return pl.pallas_call(
      body,
      grid=grid,
      in_specs=in_specs,
      out_specs=_row_spec(out_w),
      out_shape=jax.ShapeDtypeStruct((n_pad, out_w), jnp.float32),
  )(*args)


def _pad_rows(x, n):
  return jnp.pad(x, ((0, n - x.shape[0]), (0, 0)))


def _pad_edges(ei):
  src = jnp.concatenate(
      [ei[0], jnp.zeros((E_PAD - E,), jnp.int32)]).reshape(E_ROWS, 128)
  dst = jnp.concatenate(
      [ei[1], jnp.full((E_PAD - E,), SENTINEL, jnp.int32)]).reshape(
          E_ROWS, 128)
  return src, dst


@jax.jit
def _run(x_coedge, x_face, x_edge, ei_next, ei_mate, ei_to_face, ei_to_edge,
         params):
  p = params
  xco = _pad_rows(x_coedge, N_CO_PAD)
  xfa = _pad_rows(x_face, N_FACE_PAD)
  xed = _pad_rows(x_edge, N_EDGE_PAD)
  e_next = _pad_edges(ei_next)
  e_mate = _pad_edges(ei_mate)
  e_face = _pad_edges(ei_to_face)
  e_edge = _pad_edges(ei_to_edge)
  zf = jnp.zeros((16, 128), jnp.float32)
  zc_co = jnp.zeros((784,), jnp.float32)
  zc_fa = jnp.zeros((320,), jnp.float32)

  agg_co = _make_agg(N_CO_PAD, 4, 12544)
  agg_fa = _make_agg(N_FACE_PAD, 1, 5120)
  agg_ed = _make_agg(N_EDGE_PAD, 2, 12544)

  def b2(name):
    return p[name].reshape(1, 128)

  m1n = agg_co(xco, *e_next, zf, zc_co)
  m1m = agg_co(xco, *e_mate, zf, zc_co)
  m1f = agg_fa(xco, *e_face, zf, zc_fa)
  m1e = agg_ed(xco, *e_edge, zf, zc_co)

  h_co = _call_tc(
      _tc_two, N_CO_PAD, 128, [128, 128, 128],
      [(128, 128)] * 4 + [(1, 128)] * 2,
      (m1n, m1m, xco, p['W1_next_l'], p['W1_next_r'], p['W1_mate_l'],
       p['W1_mate_r'], b2('b1_next'), b2('b1_mate')))
  h_fa = _call_tc(
      _tc_one, N_FACE_PAD, 128, [128, 128],
      [(128, 128)] * 2 + [(1, 128)],
      (m1f, xfa, p['W1_to_face_l'], p['W1_to_face_r'], b2('b1_to_face')))
  h_ed = _call_tc(
      _tc_one, N_EDGE_PAD, 128, [128, 128],
      [(128, 128)] * 2 + [(1, 128)],
      (m1e, xed, p['W1_to_edge_l'], p['W1_to_edge_r'], b2('b1_to_edge')))

  m2n = agg_co(h_co, *e_next, zf, zc_co)
  m2m = agg_co(h_co, *e_mate, zf, zc_co)
  m2f = agg_fa(h_co, *e_face, zf, zc_fa)
  m2e = agg_ed(h_co, *e_edge, zf, zc_co)

  z_co = _call_tc(
      _tc_two_ro, N_CO_PAD, 256, [128, 128, 128],
      [(128, 128)] * 4 + [(1, 128)] * 2 + [(128, 256), (1, 256)],
      (m2n, m2m, h_co, p['W2_next_l'], p['W2_next_r'], p['W2_mate_l'],
       p['W2_mate_r'], b2('b2_next'), b2('b2_mate'), p['Wro_coedge'],
       p['bro_coedge'].reshape(1, 256)))
  z_fa = _call_tc(
      _tc_one_ro, N_FACE_PAD, 256, [128, 128],
      [(128, 128)] * 2 + [(1, 128), (128, 256), (1, 256)],
      (m2f, h_fa, p['W2_to_face_l'], p['W2_to_face_r'], b2('b2_to_face'),
       p['Wro_face'], p['bro_face'].reshape(1, 256)))
  z_ed = _call_tc(
      _tc_one_ro, N_EDGE_PAD, 256, [128, 128],
      [(128, 128)] * 2 + [(1, 128), (128, 256), (1, 256)],
      (m2e, h_ed, p['W2_to_edge_l'], p['W2_to_edge_r'], b2('b2_to_edge'),
       p['Wro_edge'], p['bro_edge'].reshape(1, 256)))

  return (z_co[:N_CO], z_fa[:N_FACE], z_ed[:N_EDGE])


def kernel(x_coedge, x_face, x_edge, ei_next, ei_mate, ei_to_face,
           ei_to_edge, params):
  return _run(x_coedge, x_face, x_edge, ei_next, ei_mate, ei_to_face,
              ei_to_edge, params)


# speedup vs baseline: 1.0110x; 1.0011x over previous
import jax
import jax.numpy as jnp
from jax import lax
from jax.experimental import pallas as pl
from jax.experimental.pallas import tpu as pltpu
from jax.experimental.pallas import tpu_sc as plsc

N_CO = 100000
N_FACE = 10000
N_EDGE = 50000
E = 100000
BR = 512
E_PAD = 106496
E_ROWS = E_PAD // 128
ROWS_PER_TILE = E_ROWS // 16
NBATCH = ROWS_PER_TILE
SENTINEL = 1 << 30

N_CO_PAD = 100352
N_FACE_PAD = 10240
N_EDGE_PAD = 50176


def _splat(vec, lane):
  return lax.gather(
      vec, jnp.full((16, 1), lane, jnp.int32),
      dimension_numbers=lax.GatherDimensionNumbers(
          offset_dims=(), collapsed_slice_dims=(0,), start_index_map=(0,)),
      slice_sizes=(1,), mode=lax.GatherScatterMode.PROMISE_IN_BOUNDS)


def _make_agg(n_pad, n_passes, r_half):
  chunk = 2 * r_half
  r16 = r_half // 16
  n_wchunk = r16 // 16
  assert n_passes * chunk == n_pad and r_half % 256 == 0

  mesh = plsc.VectorSubcoreMesh(
      core_axis_name="c", subcore_axis_name="s", num_cores=2, num_subcores=16)

  def body(tab, src, dst, zfeat, zcnt, out, acc, cacc, rows_v, src_i, dst_i,
           dloc, obuf, cbuf, zbuf, zcb, ones_v, *sems):
    cid = lax.axis_index("c")
    sid = lax.axis_index("s")
    pltpu.sync_copy(zfeat, zbuf)
    pltpu.sync_copy(zcnt, zcb)
    for f in range(8):
      ones_v[pl.ds(f * 16, 16)] = jnp.ones((16,), jnp.float32)

    def pass_body(p, carry):
      lo = p * chunk + cid * r_half
      for z in range(n_wchunk):
        pltpu.sync_copy(zbuf, acc.at[pl.ds(sid * r16 + z * 16, 16)])
      pltpu.sync_copy(zcb, cacc.at[pl.ds(sid * r16, r16)])
      plsc.subcore_barrier()

      def batch(b, c2):
        erow = sid * ROWS_PER_TILE + b * 4
        pltpu.sync_copy(src.at[pl.ds(erow, 4)], src_i)
        pltpu.sync_copy(dst.at[pl.ds(erow, 4)], dst_i)
        for jb in range(4):
          for v in range(8):
            d = dst_i[jb, pl.ds(v * 16, 16)]
            m = (d >= lo) & (d < lo + r_half)
            dummy = r_half + sid * 16 + 2 * v
            dloc[jb, pl.ds(v * 16, 16)] = jnp.where(m, d - lo, dummy)
          descs = [
              pltpu.async_copy(tab.at[src_i.at[jb][pl.ds(u * 16, 16)]],
                               rows_v.at[pl.ds(u * 16, 16)], sems[u])
              for u in range(8)
          ]
          for dsc in descs:
            dsc.wait()
          pltpu.sync_copy(rows_v, acc.at[dloc.at[jb]], add=True)
          pltpu.sync_copy(ones_v, cacc.at[dloc.at[jb]], add=True)
        return c2

      lax.fori_loop(0, NBATCH // 4, batch, 0)
      plsc.subcore_barrier()

      def wchunk(ch, c3):
        rbase = sid * r16 + ch * 16
        pltpu.sync_copy(acc.at[pl.ds(rbase, 16)], obuf)
        pltpu.sync_copy(cacc.at[pl.ds(rbase, 16)], cbuf)
        rcp = 1.0 / jnp.clip(cbuf[pl.ds(0, 16)], 1.0, None)
        for r in range(16):
          scal = _splat(rcp, r)
          for f in range(8):
            obuf[r, pl.ds(f * 16, 16)] = obuf[r, pl.ds(f * 16, 16)] * scal
        pltpu.sync_copy(obuf, out.at[pl.ds(lo + rbase, 16)])
        return c3

      lax.fori_loop(0, n_wchunk, wchunk, 0)
      plsc.subcore_barrier()
      return carry

    lax.fori_loop(0, n_passes, pass_body, 0)

  return pl.kernel(
      body,
      out_type=jax.ShapeDtypeStruct((n_pad, 128), jnp.float32),
      mesh=mesh,
      scratch_types=[
          pltpu.VMEM_SHARED((r_half + 256, 128), jnp.float32),
          pltpu.VMEM_SHARED((r_half + 256,), jnp.float32),
          pltpu.VMEM((128, 128), jnp.float32),
          pltpu.VMEM((4, 128), jnp.int32),
          pltpu.VMEM((4, 128), jnp.int32),
          pltpu.VMEM((4, 128), jnp.int32),
          pltpu.VMEM((16, 128), jnp.float32),
          pltpu.VMEM((16,), jnp.float32),
          pltpu.VMEM((16, 128), jnp.float32),
          pltpu.VMEM((r16,), jnp.float32),
          pltpu.VMEM((128,), jnp.float32),
      ] + [pltpu.SemaphoreType.DMA] * 8,
  )


def _dot(a, b):
  return jnp.dot(a, b, preferred_element_type=jnp.float32)


def _tc_two(mn, mm, x, wnl, wnr, wml, wmr, bn, bm, o):
  acc = (_dot(mn[:], wnl[:]) + _dot(mm[:], wml[:])
         + _dot(x[:], wnr[:] + wmr[:]) + bn[:] + bm[:])
  o[:] = jnp.maximum(acc, 0.0)


def _tc_one(s, x, wl, wr, b, o):
  acc = _dot(s[:], wl[:]) + _dot(x[:], wr[:]) + b[:]
  o[:] = jnp.maximum(acc, 0.0)


def _tc_two_ro(mn, mm, h, wnl, wnr, wml, wmr, bn, bm, wro, bro, o):
  g = jnp.maximum(
      _dot(mn[:], wnl[:]) + _dot(mm[:], wml[:])
      + _dot(h[:], wnr[:] + wmr[:]) + bn[:] + bm[:], 0.0)
  o[:] = _dot(g, wro[:]) + bro[:]


def _tc_one_ro(s, h, wl, wr, b, wro, bro, o):
  g = jnp.maximum(_dot(s[:], wl[:]) + _dot(h[:], wr[:]) + b[:], 0.0)
  o[:] = _dot(g, wro[:]) + bro[:]


def _row_spec(w):
  return pl.BlockSpec((BR, w), lambda i: (i, 0))


def _full_spec(shape):
  return pl.BlockSpec(shape, lambda i: (0, 0))


def _call_tc(body, n_pad, out_w, row_widths, full_shapes, args):
  grid = (n_pad // BR,)
  in_specs = ([_row_spec(w) for w in row_widths]
              + [_full_spec(s) for s in full_shapes])
  return pl.pallas_call(
      body,
      grid=grid,
      in_specs=in_specs,
      out_specs=_row_spec(out_w),
      out_shape=jax.ShapeDtypeStruct((n_pad, out_w), jnp.float32),
  )(*args)


def _pad_rows(x, n):
  return jnp.pad(x, ((0, n - x.shape[0]), (0, 0)))


def _pad_edges(ei):
  src = jnp.concatenate(
      [ei[0], jnp.zeros((E_PAD - E,), jnp.int32)]).reshape(E_ROWS, 128)
  dst = jnp.concatenate(
      [ei[1], jnp.full((E_PAD - E,), SENTINEL, jnp.int32)]).reshape(
          E_ROWS, 128)
  return src, dst


@jax.jit
def _run(x_coedge, x_face, x_edge, ei_next, ei_mate, ei_to_face, ei_to_edge,
         params):
  p = params
  xco = _pad_rows(x_coedge, N_CO_PAD)
  xfa = _pad_rows(x_face, N_FACE_PAD)
  xed = _pad_rows(x_edge, N_EDGE_PAD)
  e_next = _pad_edges(ei_next)
  e_mate = _pad_edges(ei_mate)
  e_face = _pad_edges(ei_to_face)
  e_edge = _pad_edges(ei_to_edge)
  zf = jnp.zeros((16, 128), jnp.float32)
  zc_co = jnp.zeros((784,), jnp.float32)
  zc_fa = jnp.zeros((320,), jnp.float32)

  agg_co = _make_agg(N_CO_PAD, 4, 12544)
  agg_fa = _make_agg(N_FACE_PAD, 1, 5120)
  agg_ed = _make_agg(N_EDGE_PAD, 2, 12544)

  def b2(name):
    return p[name].reshape(1, 128)

  m1n = agg_co(xco, *e_next, zf, zc_co)
  m1m = agg_co(xco, *e_mate, zf, zc_co)
  m1f = agg_fa(xco, *e_face, zf, zc_fa)
  m1e = agg_ed(xco, *e_edge, zf, zc_co)

  h_co = _call_tc(
      _tc_two, N_CO_PAD, 128, [128, 128, 128],
      [(128, 128)] * 4 + [(1, 128)] * 2,
      (m1n, m1m, xco, p['W1_next_l'], p['W1_next_r'], p['W1_mate_l'],
       p['W1_mate_r'], b2('b1_next'), b2('b1_mate')))
  h_fa = _call_tc(
      _tc_one, N_FACE_PAD, 128, [128, 128],
      [(128, 128)] * 2 + [(1, 128)],
      (m1f, xfa, p['W1_to_face_l'], p['W1_to_face_r'], b2('b1_to_face')))
  h_ed = _call_tc(
      _tc_one, N_EDGE_PAD, 128, [128, 128],
      [(128, 128)] * 2 + [(1, 128)],
      (m1e, xed, p['W1_to_edge_l'], p['W1_to_edge_r'], b2('b1_to_edge')))

  m2n = agg_co(h_co, *e_next, zf, zc_co)
  m2m = agg_co(h_co, *e_mate, zf, zc_co)
  m2f = agg_fa(h_co, *e_face, zf, zc_fa)
  m2e = agg_ed(h_co, *e_edge, zf, zc_co)

  z_co = _call_tc(
      _tc_two_ro, N_CO_PAD, 256, [128, 128, 128],
      [(128, 128)] * 4 + [(1, 128)] * 2 + [(128, 256), (1, 256)],
      (m2n, m2m, h_co, p['W2_next_l'], p['W2_next_r'], p['W2_mate_l'],
       p['W2_mate_r'], b2('b2_next'), b2('b2_mate'), p['Wro_coedge'],
       p['bro_coedge'].reshape(1, 256)))
  z_fa = _call_tc(
      _tc_one_ro, N_FACE_PAD, 256, [128, 128],
      [(128, 128)] * 2 + [(1, 128), (128, 256), (1, 256)],
      (m2f, h_fa, p['W2_to_face_l'], p['W2_to_face_r'], b2('b2_to_face'),
       p['Wro_face'], p['bro_face'].reshape(1, 256)))
  z_ed = _call_tc(
      _tc_one_ro, N_EDGE_PAD, 256, [128, 128],
      [(128, 128)] * 2 + [(1, 128), (128, 256), (1, 256)],
      (m2e, h_ed, p['W2_to_edge_l'], p['W2_to_edge_r'], b2('b2_to_edge'),
       p['Wro_edge'], p['bro_edge'].reshape(1, 256)))

  return (z_co[:N_CO], z_fa[:N_FACE], z_ed[:N_EDGE])


def kernel(x_coedge, x_face, x_edge, ei_next, ei_mate, ei_to_face,
           ei_to_edge, params):
  return _run(x_coedge, x_face, x_edge, ei_next, ei_mate, ei_to_face,
              ei_to_edge, params)
